# baseline (speedup 1.0000x reference)
import functools

import jax
import jax.numpy as jnp
from jax import lax
from jax.experimental import pallas as pl
from jax.experimental.pallas import tpu as pltpu
from jax.experimental.pallas import tpu_sc as plsc

VOCAB = 100000
EMB = 768
SEQ = 50
HID = 768
BATCH = 1024
LN_EPS = 1e-12

N_TOK = BATCH * SEQ

NUM_CORES = 2
NUM_SUBCORES = 16
NW = NUM_CORES * NUM_SUBCORES
PER_W = N_TOK // NW
CHUNK = 80
N_PAIRS = PER_W // (2 * CHUNK)


def _sc_gather_body(idx_hbm, table_hbm, out_hbm, idx_v,
                    buf_a, buf_b, sg_a, sg_b, ss_a, ss_b):
    wid = lax.axis_index("s") * NUM_CORES + lax.axis_index("c")
    base = wid * PER_W
    pltpu.sync_copy(idx_hbm.at[pl.ds(base, PER_W)], idx_v)

    @pl.loop(0, N_PAIRS)
    def pair(j):
        loc = pl.multiple_of(j * (2 * CHUNK), 8)
        ga = pltpu.async_copy(
            table_hbm.at[idx_v.at[pl.ds(loc, CHUNK)]], buf_a, sg_a)
        gb = pltpu.async_copy(
            table_hbm.at[idx_v.at[pl.ds(loc + CHUNK, CHUNK)]], buf_b, sg_b)
        ga.wait()
        sa = pltpu.async_copy(buf_a, out_hbm.at[pl.ds(base + loc, CHUNK)], ss_a)
        gb.wait()
        sb = pltpu.async_copy(
            buf_b, out_hbm.at[pl.ds(base + loc + CHUNK, CHUNK)], ss_b)
        sa.wait()
        sb.wait()


_sc_gather = functools.partial(
    pl.kernel,
    out_type=jax.ShapeDtypeStruct((N_TOK, EMB), jnp.float32),
    mesh=plsc.VectorSubcoreMesh(
        core_axis_name="c", subcore_axis_name="s",
        num_cores=NUM_CORES, num_subcores=NUM_SUBCORES),
    scratch_types=[
        pltpu.VMEM((PER_W,), jnp.int32),
        pltpu.VMEM((CHUNK, EMB), jnp.float32),
        pltpu.VMEM((CHUNK, EMB), jnp.float32),
        pltpu.SemaphoreType.DMA,
        pltpu.SemaphoreType.DMA,
        pltpu.SemaphoreType.DMA,
        pltpu.SemaphoreType.DMA,
    ],
)(_sc_gather_body)


def _tc_body(g_ref, pos_ref, ls_ref, lb_ref, w_ref, b_ref, o_ref):
    h = g_ref[...] + pos_ref[0]
    m = jnp.mean(h, axis=1, keepdims=True)
    hc = h - m
    v = jnp.mean(hc * hc, axis=1, keepdims=True)
    hn = hc * lax.rsqrt(v + LN_EPS)
    hn = hn * ls_ref[...] + lb_ref[...]
    o_ref[0] = (
        jnp.dot(hn.astype(jnp.bfloat16), w_ref[...].astype(jnp.bfloat16),
                preferred_element_type=jnp.float32)
        + b_ref[...]
    )


def _tc_call(gathered, pos_emb, ln_scale, ln_bias, dense_kernel, dense_bias):
    return pl.pallas_call(
        _tc_body,
        grid=(SEQ,),
        in_specs=[
            pl.BlockSpec((BATCH, EMB), lambda i: (i, 0)),
            pl.BlockSpec((1, 1, EMB), lambda i: (i, 0, 0)),
            pl.BlockSpec((1, EMB), lambda i: (0, 0)),
            pl.BlockSpec((1, EMB), lambda i: (0, 0)),
            pl.BlockSpec((EMB, HID), lambda i: (0, 0)),
            pl.BlockSpec((1, HID), lambda i: (0, 0)),
        ],
        out_specs=pl.BlockSpec((1, BATCH, HID), lambda i: (i, 0, 0)),
        out_shape=jax.ShapeDtypeStruct((SEQ, BATCH, HID), jnp.float32),
    )(gathered, pos_emb, ln_scale, ln_bias, dense_kernel, dense_bias)


def kernel(x, word_emb, pos_emb, ln_scale, ln_bias, dense_kernel, dense_bias):
    idx = x.T.reshape(-1).astype(jnp.int32)
    gathered = _sc_gather(idx, word_emb)
    out_t = _tc_call(
        gathered,
        pos_emb.reshape(SEQ, 1, EMB),
        ln_scale.reshape(1, EMB),
        ln_bias.reshape(1, EMB),
        dense_kernel,
        dense_bias.reshape(1, HID),
    )
    return out_t.transpose(1, 0, 2)
